# baseline (speedup 1.0000x reference)
import functools

import jax
import jax.numpy as jnp
from jax import lax
from jax.experimental import pallas as pl
from jax.experimental.pallas import tpu as pltpu
from jax.experimental.pallas import tpu_sc as plsc

NX = 432
NY = 496
NF = 64
NB = 4
PPB = 25000
PPB_PAD = 25088

NS = 16
WPB = 8

YROWS = 64
YROWS_L = NY - 7 * YROWS
OWN_PIX = YROWS * NX
OWN_PIX_L = YROWS_L * NX
NYT = YROWS // 8
NYT_L = YROWS_L // 8
UPIX = 8 * 128

PCHUNK = 4992
PTAIL = PPB - 5 * PCHUNK

XT_W = (128, 128, 128, NX - 384)
NXP = 512


def _body(y_hbm, x_hbm, feat_hbm, out_hbm,
          y_v, x_v, inv_v, cidx_v, cj_v, rows_v, blk_v,
          ysem, gsem, osem):
    lanes = lax.iota(jnp.int32, 16)
    i32 = jnp.int32
    wid = lax.axis_index("c") * NS + lax.axis_index("s")
    batch = wid // WPB
    k = wid % WPB
    y0 = k * YROWS
    start = k * OWN_PIX
    last = k == (WPB - 1)
    my_size = jnp.where(last, OWN_PIX_L, OWN_PIX)
    nyt = jnp.where(last, NYT_L, NYT)
    prow0 = batch * PPB
    crow0 = batch * PPB_PAD

    def init_body(i, _):
        inv_v[pl.ds(i * 16, 16)] = jnp.full((16,), -1, i32)
        return None
    lax.fori_loop(0, OWN_PIX // 16, init_body, None)

    def scan_groups(base_row, g, extra_mask_len):
        yv = y_v[pl.ds(g * 16, 16)]
        xv = x_v[pl.ds(g * 16, 16)]
        local = yv * NX + xv - start
        m = (local >= 0) & (local < my_size)
        if extra_mask_len is not None:
            m = m & (g * 16 + lanes < extra_mask_len)
        safe = jnp.clip(local, 0, OWN_PIX - 1)
        pid = base_row + g * 16 + lanes
        plsc.store_scatter(inv_v, [safe], pid, mask=m)

    def scan_chunk(c, _):
        coff = crow0 + c * PCHUNK
        pltpu.async_copy(y_hbm.at[pl.ds(coff, PCHUNK)], y_v, ysem)
        pltpu.async_copy(x_hbm.at[pl.ds(coff, PCHUNK)], x_v, ysem)
        pltpu.make_async_copy(y_hbm.at[pl.ds(coff, PCHUNK)], y_v, ysem).wait()
        pltpu.make_async_copy(x_hbm.at[pl.ds(coff, PCHUNK)], x_v, ysem).wait()

        def g_body(g, _):
            scan_groups(prow0 + c * PCHUNK, g, None)
            return None
        lax.fori_loop(0, PCHUNK // 16, g_body, None)
        return None
    lax.fori_loop(0, 5, scan_chunk, None)

    toff = crow0 + 5 * PCHUNK
    pltpu.async_copy(y_hbm.at[pl.ds(toff, 128)], y_v.at[pl.ds(0, 128)], ysem)
    pltpu.async_copy(x_hbm.at[pl.ds(toff, 128)], x_v.at[pl.ds(0, 128)], ysem)
    pltpu.make_async_copy(y_hbm.at[pl.ds(toff, 128)], y_v.at[pl.ds(0, 128)], ysem).wait()
    pltpu.make_async_copy(x_hbm.at[pl.ds(toff, 128)], x_v.at[pl.ds(0, 128)], ysem).wait()
    for g in range(3):
        scan_groups(prow0 + 5 * PCHUNK, g, PTAIL)

    def do_unit(yt, xt):
        W = XT_W[xt]
        ngx = W // 16

        def compact(r, off):
            for gx in range(ngx):
                iv = inv_v[pl.ds((yt * 8 + r) * NX + xt * 128 + gx * 16, 16)]
                m = iv >= 0
                mi = m.astype(i32)
                incl = plsc.cumsum(mi)
                pos = off + incl - 1
                plsc.store_scatter(cidx_v, [pos], iv >> 1, mask=m)
                cjval = (r * 128 + gx * 16 + lanes) | ((iv & 1) << 10)
                plsc.store_scatter(cj_v, [pos], cjval, mask=m)
                off = off + jnp.sum(mi)
            return off
        nv = lax.fori_loop(0, 8, compact, jnp.int32(0))

        plsc.store_scatter(cidx_v, [nv + lanes], batch * (PPB // 2) + lanes,
                           mask=(nv + lanes) < UPIX)
        nw = (nv + 127) // 128

        def fire_wave(w):
            base = w * 128
            ngd = (jnp.minimum(nv - base, 128) + 15) // 16

            def fire(d, _):
                pltpu.async_copy(
                    feat_hbm.at[cidx_v.at[pl.ds(base + d * 16, 16)]],
                    rows_v.at[pl.ds(d * 16, 16), :], gsem)
                return None
            lax.fori_loop(0, ngd, fire, None)
            return ngd

        ngd0 = fire_wave(jnp.int32(0))

        uglob = yt * 4 + xt

        @pl.when(uglob >= 1)
        def _():
            pltpu.make_async_copy(
                blk_v, out_hbm.at[batch, :, pl.ds(y0, 8), pl.ds(0, 128)],
                osem).wait()

        def zero_row(f, _):
            for r in range(8):
                for g2 in range(8):
                    blk_v[f, r, pl.ds(g2 * 16, 16)] = jnp.zeros((16,), jnp.float32)
            return None
        lax.fori_loop(0, NF, zero_row, None)

        def wave(w, ngd_w):
            base = w * 128

            def drain(d, _):
                pltpu.make_async_copy(
                    feat_hbm.at[cidx_v.at[pl.ds(base + d * 16, 16)]],
                    rows_v.at[pl.ds(d * 16, 16), :], gsem).wait()
                return None
            lax.fori_loop(0, ngd_w, drain, None)

            def sgroup(t2, _):
                for l in range(16):
                    slot = base + t2 * 16 + l
                    mv = jnp.broadcast_to(slot < nv, (16,))
                    jbf = plsc.load_gather(cj_v, [jnp.full((16,), slot, i32)])
                    xl = jbf & 127
                    rr = (jbf >> 7) & 7
                    parcol = ((jbf >> 10) & 1) * 64
                    for q in range(4):
                        rv = plsc.load_gather(
                            rows_v,
                            [jnp.full((16,), slot - base, i32),
                             parcol + q * 16 + lanes])
                        plsc.store_scatter(blk_v, [q * 16 + lanes, rr, xl],
                                           rv, mask=mv)
                return None
            lax.fori_loop(0, ngd_w, sgroup, None)

        wave(jnp.int32(0), ngd0)

        def later_wave(w, _):
            ngd_w = fire_wave(w)
            wave(w, ngd_w)
            return None
        lax.fori_loop(1, nw, later_wave, None)

        pltpu.async_copy(
            blk_v,
            out_hbm.at[batch, :, pl.ds(y0 + yt * 8, 8), pl.ds(xt * 128, 128)],
            osem)

    def yt_body(yt, _):
        for xt in range(4):
            do_unit(yt, xt)
        return None
    lax.fori_loop(0, nyt, yt_body, None)

    pltpu.make_async_copy(
        blk_v, out_hbm.at[batch, :, pl.ds(y0, 8), pl.ds(0, 128)], osem).wait()


_scatter = functools.partial(
    pl.kernel,
    out_type=jax.ShapeDtypeStruct((NB, NF, NY, NXP), jnp.float32),
    mesh=plsc.VectorSubcoreMesh(core_axis_name="c", subcore_axis_name="s"),
    scratch_types=[
        pltpu.VMEM((PCHUNK,), jnp.int32),
        pltpu.VMEM((PCHUNK,), jnp.int32),
        pltpu.VMEM((OWN_PIX,), jnp.int32),
        pltpu.VMEM((UPIX,), jnp.int32),
        pltpu.VMEM((UPIX,), jnp.int32),
        pltpu.VMEM((128, 128), jnp.float32),
        pltpu.VMEM((NF, 8, 128), jnp.float32),
        pltpu.SemaphoreType.DMA,
        pltpu.SemaphoreType.DMA,
        pltpu.SemaphoreType.DMA,
    ],
    compiler_params=pltpu.CompilerParams(needs_layout_passes=False),
)(_body)


def _trunc_body(i_ref, o_ref):
    o_ref[...] = i_ref[:, :, :, :NX]


_truncate = pl.pallas_call(
    _trunc_body,
    grid=(NB, NY // 8),
    in_specs=[pl.BlockSpec((1, NF, 8, NXP), lambda b, t: (b, 0, t, 0))],
    out_specs=pl.BlockSpec((1, NF, 8, NX), lambda b, t: (b, 0, t, 0)),
    out_shape=jax.ShapeDtypeStruct((NB, NF, NY, NX), jnp.float32),
    compiler_params=pltpu.CompilerParams(
        dimension_semantics=("parallel", "arbitrary")),
)


@jax.jit
def kernel(voxel_coords, pillar_features):
    vc = voxel_coords.astype(jnp.int32)
    y = jnp.pad(vc[:, 2].reshape(NB, PPB),
                ((0, 0), (0, PPB_PAD - PPB))).reshape(-1)
    x = jnp.pad(vc[:, 3].reshape(NB, PPB),
                ((0, 0), (0, PPB_PAD - PPB))).reshape(-1)
    feat2 = pillar_features.reshape(PPB * 2, 128)
    return _truncate(_scatter(y, x, feat2))


# speedup vs baseline: 1.0063x; 1.0063x over previous
import functools

import jax
import jax.numpy as jnp
from jax import lax
from jax.experimental import pallas as pl
from jax.experimental.pallas import tpu as pltpu
from jax.experimental.pallas import tpu_sc as plsc

NX = 432
NY = 496
NF = 64
NB = 4
PPB = 25000
PPB_PAD = 25088

NS = 16
WPB = 8

YROWS = 64
YROWS_L = NY - 7 * YROWS
OWN_PIX = YROWS * NX
OWN_PIX_L = YROWS_L * NX
NYT = YROWS // 8
NYT_L = YROWS_L // 8
UPIX = 8 * 128

PCHUNK = 4992
PTAIL = PPB - 5 * PCHUNK

XT_W = (128, 128, 128, NX - 384)
NXP = 512


def _body(y_hbm, x_hbm, feat_hbm, out_hbm,
          y_v, x_v, inv_v, cidx_v, cj0_v, cj1_v, rows_v, blk_v,
          ysem, gsem, osem):
    lanes = lax.iota(jnp.int32, 16)
    i32 = jnp.int32
    wid = lax.axis_index("c") * NS + lax.axis_index("s")
    batch = wid // WPB
    k = wid % WPB
    y0 = k * YROWS
    start = k * OWN_PIX
    last = k == (WPB - 1)
    my_size = jnp.where(last, OWN_PIX_L, OWN_PIX)
    nyt = jnp.where(last, NYT_L, NYT)
    prow0 = batch * PPB
    crow0 = batch * PPB_PAD

    def init_body(i, _):
        inv_v[pl.ds(i * 16, 16)] = jnp.full((16,), -1, i32)
        return None
    lax.fori_loop(0, OWN_PIX // 16, init_body, None)

    def scan_groups(base_row, g, extra_mask_len):
        yv = y_v[pl.ds(g * 16, 16)]
        xv = x_v[pl.ds(g * 16, 16)]
        local = yv * NX + xv - start
        m = (local >= 0) & (local < my_size)
        if extra_mask_len is not None:
            m = m & (g * 16 + lanes < extra_mask_len)
        safe = jnp.clip(local, 0, OWN_PIX - 1)
        pid = base_row + g * 16 + lanes
        plsc.store_scatter(inv_v, [safe], pid, mask=m)

    def scan_chunk(c, _):
        coff = crow0 + c * PCHUNK
        pltpu.async_copy(y_hbm.at[pl.ds(coff, PCHUNK)], y_v, ysem)
        pltpu.async_copy(x_hbm.at[pl.ds(coff, PCHUNK)], x_v, ysem)
        pltpu.make_async_copy(y_hbm.at[pl.ds(coff, PCHUNK)], y_v, ysem).wait()
        pltpu.make_async_copy(x_hbm.at[pl.ds(coff, PCHUNK)], x_v, ysem).wait()

        def g_body(g, _):
            scan_groups(prow0 + c * PCHUNK, g, None)
            return None
        lax.fori_loop(0, PCHUNK // 16, g_body, None)
        return None
    lax.fori_loop(0, 5, scan_chunk, None)

    toff = crow0 + 5 * PCHUNK
    pltpu.async_copy(y_hbm.at[pl.ds(toff, 128)], y_v.at[pl.ds(0, 128)], ysem)
    pltpu.async_copy(x_hbm.at[pl.ds(toff, 128)], x_v.at[pl.ds(0, 128)], ysem)
    pltpu.make_async_copy(y_hbm.at[pl.ds(toff, 128)], y_v.at[pl.ds(0, 128)], ysem).wait()
    pltpu.make_async_copy(x_hbm.at[pl.ds(toff, 128)], x_v.at[pl.ds(0, 128)], ysem).wait()
    for g in range(3):
        scan_groups(prow0 + 5 * PCHUNK, g, PTAIL)

    def zero_row(f, _):
        for r in range(8):
            for g2 in range(8):
                blk_v[f, r, pl.ds(g2 * 16, 16)] = jnp.zeros((16,), jnp.float32)
        return None
    lax.fori_loop(0, NF, zero_row, None)

    def do_unit(yt, xt, cj_v, cjp_v, nvp):
        W = XT_W[xt]
        ngx = W // 16

        def compact(r, off):
            for gx in range(ngx):
                iv = inv_v[pl.ds((yt * 8 + r) * NX + xt * 128 + gx * 16, 16)]
                m = iv >= 0
                mi = m.astype(i32)
                incl = plsc.cumsum(mi)
                pos = off + incl - 1
                plsc.store_scatter(cidx_v, [pos], iv >> 1, mask=m)
                cjval = (r * 128 + gx * 16 + lanes) | ((iv & 1) << 10)
                plsc.store_scatter(cj_v, [pos], cjval, mask=m)
                off = off + jnp.sum(mi)
            return off
        nv = lax.fori_loop(0, 8, compact, jnp.int32(0))

        plsc.store_scatter(cidx_v, [nv + lanes], batch * (PPB // 2) + lanes,
                           mask=(nv + lanes) < UPIX)
        nw = (nv + 127) // 128

        def fire_wave(w):
            base = w * 128
            ngd = (jnp.minimum(nv - base, 128) + 15) // 16

            def fire(d, _):
                pltpu.async_copy(
                    feat_hbm.at[cidx_v.at[pl.ds(base + d * 16, 16)]],
                    rows_v.at[pl.ds(d * 16, 16), :], gsem)
                return None
            lax.fori_loop(0, ngd, fire, None)
            return ngd

        ngd0 = fire_wave(jnp.int32(0))

        uglob = yt * 4 + xt

        @pl.when(uglob >= 1)
        def _():
            pltpu.make_async_copy(
                blk_v, out_hbm.at[batch, :, pl.ds(y0, 8), pl.ds(0, 128)],
                osem).wait()

        zeros16 = jnp.zeros((16,), jnp.float32)

        def rgroup(t2, _):
            for l in range(16):
                slot = t2 * 16 + l
                mv = jnp.broadcast_to(slot < nvp, (16,))
                jbf = plsc.load_gather(cjp_v, [jnp.full((16,), slot, i32)])
                xl = jbf & 127
                rr = (jbf >> 7) & 7
                for q in range(4):
                    plsc.store_scatter(blk_v, [q * 16 + lanes, rr, xl],
                                       zeros16, mask=mv)
            return None
        lax.fori_loop(0, (nvp + 15) // 16, rgroup, None)

        def wave(w, ngd_w):
            base = w * 128

            def drain(d, _):
                pltpu.make_async_copy(
                    feat_hbm.at[cidx_v.at[pl.ds(base + d * 16, 16)]],
                    rows_v.at[pl.ds(d * 16, 16), :], gsem).wait()
                return None
            lax.fori_loop(0, ngd_w, drain, None)

            def sgroup(t2, _):
                for l in range(16):
                    slot = base + t2 * 16 + l
                    mv = jnp.broadcast_to(slot < nv, (16,))
                    jbf = plsc.load_gather(cj_v, [jnp.full((16,), slot, i32)])
                    xl = jbf & 127
                    rr = (jbf >> 7) & 7
                    parcol = ((jbf >> 10) & 1) * 64
                    for q in range(4):
                        rv = plsc.load_gather(
                            rows_v,
                            [jnp.full((16,), slot - base, i32),
                             parcol + q * 16 + lanes])
                        plsc.store_scatter(blk_v, [q * 16 + lanes, rr, xl],
                                           rv, mask=mv)
                return None
            lax.fori_loop(0, ngd_w, sgroup, None)

        wave(jnp.int32(0), ngd0)

        def later_wave(w, _):
            ngd_w = fire_wave(w)
            wave(w, ngd_w)
            return None
        lax.fori_loop(1, nw, later_wave, None)

        pltpu.async_copy(
            blk_v,
            out_hbm.at[batch, :, pl.ds(y0 + yt * 8, 8), pl.ds(xt * 128, 128)],
            osem)
        return nv

    def yt_body(yt, nvp):
        for xt in range(4):
            cur, prev = (cj0_v, cj1_v) if xt % 2 == 0 else (cj1_v, cj0_v)
            nvp = do_unit(yt, xt, cur, prev, nvp)
        return nvp
    lax.fori_loop(0, nyt, yt_body, jnp.int32(0))

    pltpu.make_async_copy(
        blk_v, out_hbm.at[batch, :, pl.ds(y0, 8), pl.ds(0, 128)], osem).wait()


_scatter = functools.partial(
    pl.kernel,
    out_type=jax.ShapeDtypeStruct((NB, NF, NY, NXP), jnp.float32),
    mesh=plsc.VectorSubcoreMesh(core_axis_name="c", subcore_axis_name="s"),
    scratch_types=[
        pltpu.VMEM((PCHUNK,), jnp.int32),
        pltpu.VMEM((PCHUNK,), jnp.int32),
        pltpu.VMEM((OWN_PIX,), jnp.int32),
        pltpu.VMEM((UPIX,), jnp.int32),
        pltpu.VMEM((UPIX,), jnp.int32),
        pltpu.VMEM((UPIX,), jnp.int32),
        pltpu.VMEM((128, 128), jnp.float32),
        pltpu.VMEM((NF, 8, 128), jnp.float32),
        pltpu.SemaphoreType.DMA,
        pltpu.SemaphoreType.DMA,
        pltpu.SemaphoreType.DMA,
    ],
    compiler_params=pltpu.CompilerParams(needs_layout_passes=False),
)(_body)


def _trunc_body(i_ref, o_ref):
    o_ref[...] = i_ref[:, :, :, :NX]


_truncate = pl.pallas_call(
    _trunc_body,
    grid=(NB, NY // 16),
    in_specs=[pl.BlockSpec((1, NF, 16, NXP), lambda b, t: (b, 0, t, 0))],
    out_specs=pl.BlockSpec((1, NF, 16, NX), lambda b, t: (b, 0, t, 0)),
    out_shape=jax.ShapeDtypeStruct((NB, NF, NY, NX), jnp.float32),
)


@jax.jit
def kernel(voxel_coords, pillar_features):
    vc = voxel_coords.astype(jnp.int32)
    y = jnp.pad(vc[:, 2].reshape(NB, PPB),
                ((0, 0), (0, PPB_PAD - PPB))).reshape(-1)
    x = jnp.pad(vc[:, 3].reshape(NB, PPB),
                ((0, 0), (0, PPB_PAD - PPB))).reshape(-1)
    feat2 = pillar_features.reshape(PPB * 2, 128)
    return _truncate(_scatter(y, x, feat2))


# speedup vs baseline: 1.0897x; 1.0829x over previous
import functools

import jax
import jax.numpy as jnp
from jax import lax
from jax.experimental import pallas as pl
from jax.experimental.pallas import tpu as pltpu
from jax.experimental.pallas import tpu_sc as plsc

NX = 432
NY = 496
NF = 64
NB = 4
PPB = 25000
PPB_PAD = 25088

NS = 16
WPB = 8

YROWS = 64
YROWS_L = NY - 7 * YROWS
OWN_PIX = YROWS * NX
OWN_PIX_L = YROWS_L * NX
NYT = YROWS // 8
NYT_L = YROWS_L // 8
UPIX = 8 * 128

PCHUNK = 4992
PTAIL = PPB - 5 * PCHUNK

XT_W = (128, 128, 128, NX - 384)
NXP = 512


def _body(y_hbm, x_hbm, feat_hbm, out_hbm,
          y_v, x_v, inv_v, cidx_v, cj0_v, cj1_v, rows_v, blk_v,
          ysem, gsem, osem):
    lanes = lax.iota(jnp.int32, 16)
    i32 = jnp.int32
    wid = lax.axis_index("c") * NS + lax.axis_index("s")
    batch = wid // WPB
    k = wid % WPB
    y0 = k * YROWS
    start = k * OWN_PIX
    last = k == (WPB - 1)
    my_size = jnp.where(last, OWN_PIX_L, OWN_PIX)
    nyt = jnp.where(last, NYT_L, NYT)
    prow0 = batch * PPB
    crow0 = batch * PPB_PAD

    def init_body(i, _):
        inv_v[pl.ds(i * 16, 16)] = jnp.full((16,), -1, i32)
        return None
    lax.fori_loop(0, OWN_PIX // 16, init_body, None)

    def scan_groups(base_row, g, extra_mask_len):
        yv = y_v[pl.ds(g * 16, 16)]
        xv = x_v[pl.ds(g * 16, 16)]
        local = yv * NX + xv - start
        m = (local >= 0) & (local < my_size)
        if extra_mask_len is not None:
            m = m & (g * 16 + lanes < extra_mask_len)
        safe = jnp.clip(local, 0, OWN_PIX - 1)
        pid = base_row + g * 16 + lanes
        plsc.store_scatter(inv_v, [safe], pid, mask=m)

    def scan_chunk(c, _):
        coff = crow0 + c * PCHUNK
        pltpu.async_copy(y_hbm.at[pl.ds(coff, PCHUNK)], y_v, ysem)
        pltpu.async_copy(x_hbm.at[pl.ds(coff, PCHUNK)], x_v, ysem)
        pltpu.make_async_copy(y_hbm.at[pl.ds(coff, PCHUNK)], y_v, ysem).wait()
        pltpu.make_async_copy(x_hbm.at[pl.ds(coff, PCHUNK)], x_v, ysem).wait()

        def g_body(g, _):
            scan_groups(prow0 + c * PCHUNK, g, None)
            return None
        lax.fori_loop(0, PCHUNK // 16, g_body, None)
        return None
    lax.fori_loop(0, 5, scan_chunk, None)

    toff = crow0 + 5 * PCHUNK
    pltpu.async_copy(y_hbm.at[pl.ds(toff, 128)], y_v.at[pl.ds(0, 128)], ysem)
    pltpu.async_copy(x_hbm.at[pl.ds(toff, 128)], x_v.at[pl.ds(0, 128)], ysem)
    pltpu.make_async_copy(y_hbm.at[pl.ds(toff, 128)], y_v.at[pl.ds(0, 128)], ysem).wait()
    pltpu.make_async_copy(x_hbm.at[pl.ds(toff, 128)], x_v.at[pl.ds(0, 128)], ysem).wait()
    for g in range(3):
        scan_groups(prow0 + 5 * PCHUNK, g, PTAIL)

    def do_unit(yt, xt, cj_v):
        W = XT_W[xt]
        ngx = W // 16

        def compact(r, off):
            for gx in range(ngx):
                iv = inv_v[pl.ds((yt * 8 + r) * NX + xt * 128 + gx * 16, 16)]
                m = iv >= 0
                mi = m.astype(i32)
                incl = plsc.cumsum(mi)
                pos = off + incl - 1
                plsc.store_scatter(cidx_v, [pos], iv >> 1, mask=m)
                cjval = (r * 128 + gx * 16 + lanes) | ((iv & 1) << 10)
                plsc.store_scatter(cj_v, [pos], cjval, mask=m)
                off = off + jnp.sum(mi)
            return off
        nv = lax.fori_loop(0, 8, compact, jnp.int32(0))

        plsc.store_scatter(cidx_v, [nv + lanes], batch * (PPB // 2) + lanes,
                           mask=(nv + lanes) < UPIX)
        nw = (nv + 127) // 128

        def fire_wave(w):
            base = w * 128
            ngd = (jnp.minimum(nv - base, 128) + 15) // 16

            def fire(d, _):
                pltpu.async_copy(
                    feat_hbm.at[cidx_v.at[pl.ds(base + d * 16, 16)]],
                    rows_v.at[pl.ds(d * 16, 16), :], gsem)
                return None
            lax.fori_loop(0, ngd, fire, None)
            return ngd

        ngd0 = fire_wave(jnp.int32(0))

        uglob = yt * 4 + xt

        @pl.when(uglob >= 1)
        def _():
            pltpu.make_async_copy(
                blk_v, out_hbm.at[batch, :, pl.ds(y0, 8), pl.ds(0, 128)],
                osem).wait()

        def zero_row(f, _):
            for r in range(8):
                for g2 in range(8):
                    blk_v[f, r, pl.ds(g2 * 16, 16)] = jnp.zeros((16,), jnp.float32)
            return None
        lax.fori_loop(0, NF, zero_row, None)

        def wave(w, ngd_w):
            base = w * 128

            def drain(d, _):
                pltpu.make_async_copy(
                    feat_hbm.at[cidx_v.at[pl.ds(base + d * 16, 16)]],
                    rows_v.at[pl.ds(d * 16, 16), :], gsem).wait()
                return None
            lax.fori_loop(0, ngd_w, drain, None)

            def sgroup(t2, _):
                for l in range(16):
                    slot = base + t2 * 16 + l
                    mv = jnp.broadcast_to(slot < nv, (16,))
                    jbf = plsc.load_gather(cj_v, [jnp.full((16,), slot, i32)])
                    xl = jbf & 127
                    rr = (jbf >> 7) & 7
                    parcol = ((jbf >> 10) & 1) * 64
                    for q in range(4):
                        rv = plsc.load_gather(
                            rows_v,
                            [jnp.full((16,), slot - base, i32),
                             parcol + q * 16 + lanes])
                        plsc.store_scatter(blk_v, [q * 16 + lanes, rr, xl],
                                           rv, mask=mv)
                return None
            lax.fori_loop(0, ngd_w, sgroup, None)

        wave(jnp.int32(0), ngd0)

        def later_wave(w, _):
            ngd_w = fire_wave(w)
            wave(w, ngd_w)
            return None
        lax.fori_loop(1, nw, later_wave, None)

        pltpu.async_copy(
            blk_v,
            out_hbm.at[batch, :, pl.ds(y0 + yt * 8, 8), pl.ds(xt * 128, 128)],
            osem)

    def yt_body(yt, _):
        for xt in range(4):
            do_unit(yt, xt, cj0_v if xt % 2 == 0 else cj1_v)
        return None
    lax.fori_loop(0, nyt, yt_body, None)

    pltpu.make_async_copy(
        blk_v, out_hbm.at[batch, :, pl.ds(y0, 8), pl.ds(0, 128)], osem).wait()


_scatter = functools.partial(
    pl.kernel,
    out_type=jax.ShapeDtypeStruct((NB, NF, NY, NXP), jnp.float32),
    mesh=plsc.VectorSubcoreMesh(core_axis_name="c", subcore_axis_name="s"),
    scratch_types=[
        pltpu.VMEM((PCHUNK,), jnp.int32),
        pltpu.VMEM((PCHUNK,), jnp.int32),
        pltpu.VMEM((OWN_PIX,), jnp.int32),
        pltpu.VMEM((UPIX,), jnp.int32),
        pltpu.VMEM((UPIX,), jnp.int32),
        pltpu.VMEM((UPIX,), jnp.int32),
        pltpu.VMEM((128, 128), jnp.float32),
        pltpu.VMEM((NF, 8, 128), jnp.float32),
        pltpu.SemaphoreType.DMA,
        pltpu.SemaphoreType.DMA,
        pltpu.SemaphoreType.DMA,
    ],
    compiler_params=pltpu.CompilerParams(needs_layout_passes=False),
)(_body)


def _trunc_body(i_ref, o_ref):
    o_ref[...] = i_ref[:, :, :, :NX]


_truncate = pl.pallas_call(
    _trunc_body,
    grid=(NB, NY // 16),
    in_specs=[pl.BlockSpec((1, NF, 16, NXP), lambda b, t: (b, 0, t, 0))],
    out_specs=pl.BlockSpec((1, NF, 16, NX), lambda b, t: (b, 0, t, 0)),
    out_shape=jax.ShapeDtypeStruct((NB, NF, NY, NX), jnp.float32),
)


@jax.jit
def kernel(voxel_coords, pillar_features):
    vc = voxel_coords.astype(jnp.int32)
    y = jnp.pad(vc[:, 2].reshape(NB, PPB),
                ((0, 0), (0, PPB_PAD - PPB))).reshape(-1)
    x = jnp.pad(vc[:, 3].reshape(NB, PPB),
                ((0, 0), (0, PPB_PAD - PPB))).reshape(-1)
    feat2 = pillar_features.reshape(PPB * 2, 128)
    return _truncate(_scatter(y, x, feat2))
